# Initial kernel scaffold; baseline (speedup 1.0000x reference)
import jax
import jax.numpy as jnp
from jax.experimental import pallas as pl


def kernel(text1, text2, graph_data, scene_text, time_data, pretrained_emb, node_emb, time_w, time_b, W, b):
    raise NotImplementedError("write your pallas kernel here")


# speedup vs baseline: 1.0529x; 1.0529x over previous
import functools

import jax
import jax.numpy as jnp
from jax import lax
from jax.experimental import pallas as pl
from jax.experimental.pallas import tpu as pltpu
from jax.experimental.pallas import tpu_sc as plsc

B = 16384
VOCAB = 100000
NODE_NUM = 10000
TEXT_DIM = 64
TIME_DIM = 32

_FOLD_GRID = 10
_ROWS = VOCAB // _FOLD_GRID

_INV_PI = 0.31830987334251404
_PI_HI = 3.1415927410125732
_PI_LO = -8.742277657347586e-08
_COS_COEFS = (1.9907545e-09, -2.7524447e-07, 2.4801026e-05, -0.0013888883,
              0.041666668, -0.5, 1.0)


def _fast_cos(x):
    k = jnp.floor(x * _INV_PI + 0.5)
    r = (x - k * _PI_HI) - k * _PI_LO
    r2 = r * r
    c = jnp.full_like(r2, _COS_COEFS[0])
    for cc in _COS_COEFS[1:]:
        c = c * r2 + cc
    odd = jnp.bitwise_and(k.astype(jnp.int32), 1)
    return jnp.where(odd == 0, c, -c)


def _fold_body(emb_ref, node_ref, time_ref, tw_ref, tb_ref, W_ref, b_ref,
               tt_ref, g2_ref, tp_ref):
    eb = emb_ref[...]
    wt = jnp.concatenate((W_ref[0:64, :], W_ref[64:128, :]), axis=1)
    tt_ref[...] = jnp.dot(eb, wt, preferred_element_type=jnp.float32)

    @pl.when(pl.program_id(0) == 0)
    def _():
        g2_ref[...] = jnp.dot(node_ref[...], W_ref[128:192, :],
                              preferred_element_type=jnp.float32)
        te = _fast_cos(time_ref[...] * tw_ref[...] + tb_ref[...])
        tp_ref[...] = jnp.dot(te, W_ref[192:224, :],
                              preferred_element_type=jnp.float32) + b_ref[...]


_fold = pl.pallas_call(
    _fold_body,
    grid=(_FOLD_GRID,),
    in_specs=[
        pl.BlockSpec((_ROWS, TEXT_DIM), lambda i: (i, 0)),
        pl.BlockSpec((NODE_NUM, TEXT_DIM), lambda i: (0, 0)),
        pl.BlockSpec((B, 1), lambda i: (0, 0)),
        pl.BlockSpec((1, TIME_DIM), lambda i: (0, 0)),
        pl.BlockSpec((1, TIME_DIM), lambda i: (0, 0)),
        pl.BlockSpec((2 * TEXT_DIM + TEXT_DIM + TIME_DIM, 2), lambda i: (0, 0)),
        pl.BlockSpec((1, 2), lambda i: (0, 0)),
    ],
    out_specs=[
        pl.BlockSpec((_ROWS, 4), lambda i: (i, 0)),
        pl.BlockSpec((NODE_NUM, 2), lambda i: (0, 0)),
        pl.BlockSpec((B, 2), lambda i: (0, 0)),
    ],
    out_shape=[
        jax.ShapeDtypeStruct((VOCAB, 4), jnp.float32),
        jax.ShapeDtypeStruct((NODE_NUM, 2), jnp.float32),
        jax.ShapeDtypeStruct((B, 2), jnp.float32),
    ],
)


@functools.lru_cache(maxsize=1)
def _make_sc_gather():
    info = plsc.get_sparse_core_info()
    nc, ns = info.num_cores, info.num_subcores
    nw = nc * ns
    epw = B * 2 // nw
    nchunks = epw // 128
    ngroups = epw // 16
    mesh = plsc.VectorSubcoreMesh(core_axis_name="c", subcore_axis_name="s",
                                  num_cores=nc)

    @functools.partial(
        pl.kernel,
        mesh=mesh,
        out_type=jax.ShapeDtypeStruct((B * 2,), jnp.float32),
        scratch_types=[
            pltpu.VMEM((3, nchunks, 128), jnp.int32),
            pltpu.VMEM((epw,), jnp.float32),
            pltpu.VMEM((epw,), jnp.float32),
            pltpu.VMEM((epw,), jnp.float32),
            pltpu.VMEM((epw,), jnp.float32),
            pltpu.VMEM((epw,), jnp.float32),
            pltpu.SemaphoreType.DMA,
        ],
    )
    def sc_k(idx_hbm, tt_hbm, g2_hbm, tp_hbm, out_hbm,
             idx_v, r1_v, r2_v, rg_v, tp_v, out_v, sem):
        wid = lax.axis_index("s") * nc + lax.axis_index("c")
        base = wid * epw
        pltpu.sync_copy(idx_hbm.at[wid], idx_v)
        pltpu.sync_copy(tp_hbm.at[pl.ds(base, epw)], tp_v)
        copies = []
        for j in range(nchunks):
            dst = pl.ds(j * 128, 128)
            copies.append(pltpu.async_copy(tt_hbm.at[idx_v.at[0, j]],
                                           r1_v.at[dst], sem))
            copies.append(pltpu.async_copy(tt_hbm.at[idx_v.at[1, j]],
                                           r2_v.at[dst], sem))
            copies.append(pltpu.async_copy(g2_hbm.at[idx_v.at[2, j]],
                                           rg_v.at[dst], sem))
        for c in copies:
            c.wait()
        for g in range(ngroups):
            s = pl.ds(g * 16, 16)
            out_v[s] = r1_v[s] + r2_v[s] + rg_v[s] + tp_v[s]
        pltpu.sync_copy(out_v, out_hbm.at[pl.ds(base, epw)])

    return sc_k, nw, nchunks


def kernel(text1, text2, graph_data, scene_text, time_data, pretrained_emb,
           node_emb, time_w, time_b, W, b):
    tt, g2, tp = _fold(pretrained_emb, node_emb, time_data,
                       time_w.reshape(1, -1), time_b.reshape(1, -1),
                       W, b.reshape(1, -1))
    sc_k, nw, nchunks = _make_sc_gather()
    t1 = text1.astype(jnp.int32)
    t2 = text2.astype(jnp.int32)
    g = graph_data.astype(jnp.int32)
    idx1 = jnp.stack([4 * t1, 4 * t1 + 1], axis=-1).reshape(-1)
    idx2 = jnp.stack([4 * t2 + 2, 4 * t2 + 3], axis=-1).reshape(-1)
    idxg = jnp.stack([2 * g, 2 * g + 1], axis=-1).reshape(-1)
    idx = jnp.stack([idx1, idx2, idxg], axis=0)
    idx = idx.reshape(3, nw, nchunks, 128).transpose(1, 0, 2, 3)
    out_flat = sc_k(idx, tt.reshape(-1), g2.reshape(-1), tp.reshape(-1))
    return out_flat.reshape(B, 2)


# speedup vs baseline: 1.4269x; 1.3552x over previous
import functools

import jax
import jax.numpy as jnp
from jax import lax
from jax.experimental import pallas as pl
from jax.experimental.pallas import tpu as pltpu
from jax.experimental.pallas import tpu_sc as plsc

B = 16384
VOCAB = 100000
NODE_NUM = 10000
TEXT_DIM = 64
TIME_DIM = 32

_FOLD_GRID = 10
_ROWS = VOCAB // _FOLD_GRID

_INV_PI = 0.31830987334251404
_PI_HI = 3.1415927410125732
_PI_LO = -8.742277657347586e-08
_COS_COEFS = (1.9907545e-09, -2.7524447e-07, 2.4801026e-05, -0.0013888883,
              0.041666668, -0.5, 1.0)


def _fast_cos(x):
    k = jnp.floor(x * _INV_PI + 0.5)
    r = (x - k * _PI_HI) - k * _PI_LO
    r2 = r * r
    c = jnp.full_like(r2, _COS_COEFS[0])
    for cc in _COS_COEFS[1:]:
        c = c * r2 + cc
    odd = jnp.bitwise_and(k.astype(jnp.int32), 1)
    return jnp.where(odd == 0, c, -c)


def _fold_body(emb_ref, node_ref, time_ref, tw_ref, tb_ref, W_ref, b_ref,
               tt_ref, g2_ref, tp0_ref, tp1_ref):
    eb = emb_ref[...]
    wt = jnp.concatenate((W_ref[0:64, :], W_ref[64:128, :]), axis=1)
    tt_ref[...] = jnp.dot(eb, wt, preferred_element_type=jnp.float32)

    @pl.when(pl.program_id(0) == 0)
    def _():
        g2_ref[...] = jnp.dot(node_ref[...], W_ref[128:192, :],
                              preferred_element_type=jnp.float32)
        te = _fast_cos(time_ref[...] * tw_ref[...] + tb_ref[...])
        dn = (((1,), (1,)), ((), ()))
        tp0_ref[...] = lax.dot_general(
            W_ref[192:224, 0:1].T, te, dn,
            preferred_element_type=jnp.float32) + b_ref[0, 0]
        tp1_ref[...] = lax.dot_general(
            W_ref[192:224, 1:2].T, te, dn,
            preferred_element_type=jnp.float32) + b_ref[0, 1]


_fold = pl.pallas_call(
    _fold_body,
    grid=(_FOLD_GRID,),
    in_specs=[
        pl.BlockSpec((_ROWS, TEXT_DIM), lambda i: (i, 0)),
        pl.BlockSpec((NODE_NUM, TEXT_DIM), lambda i: (0, 0)),
        pl.BlockSpec((B, 1), lambda i: (0, 0)),
        pl.BlockSpec((1, TIME_DIM), lambda i: (0, 0)),
        pl.BlockSpec((1, TIME_DIM), lambda i: (0, 0)),
        pl.BlockSpec((2 * TEXT_DIM + TEXT_DIM + TIME_DIM, 2), lambda i: (0, 0)),
        pl.BlockSpec((1, 2), lambda i: (0, 0)),
    ],
    out_specs=[
        pl.BlockSpec((_ROWS, 4), lambda i: (i, 0)),
        pl.BlockSpec((NODE_NUM, 2), lambda i: (0, 0)),
        pl.BlockSpec((1, B), lambda i: (0, 0)),
        pl.BlockSpec((1, B), lambda i: (0, 0)),
    ],
    out_shape=[
        jax.ShapeDtypeStruct((VOCAB, 4), jnp.float32),
        jax.ShapeDtypeStruct((NODE_NUM, 2), jnp.float32),
        jax.ShapeDtypeStruct((1, B), jnp.float32),
        jax.ShapeDtypeStruct((1, B), jnp.float32),
    ],
)


@functools.lru_cache(maxsize=1)
def _make_sc_gather():
    info = plsc.get_sparse_core_info()
    nc, ns = info.num_cores, info.num_subcores
    nw = nc * ns
    bpw = B // nw
    nchunks = bpw // 128
    ngroups = bpw // 16
    mesh = plsc.VectorSubcoreMesh(core_axis_name="c", subcore_axis_name="s",
                                  num_cores=nc)

    @functools.partial(
        pl.kernel,
        mesh=mesh,
        out_type=jax.ShapeDtypeStruct((2 * B,), jnp.float32),
        scratch_types=[
            pltpu.VMEM((bpw,), jnp.int32),
            pltpu.VMEM((bpw,), jnp.int32),
            pltpu.VMEM((bpw,), jnp.int32),
            [pltpu.VMEM((bpw,), jnp.int32) for _ in range(6)],
            [pltpu.VMEM((bpw,), jnp.float32) for _ in range(6)],
            pltpu.VMEM((bpw,), jnp.float32),
            pltpu.VMEM((bpw,), jnp.float32),
            pltpu.VMEM((bpw,), jnp.float32),
            pltpu.VMEM((bpw,), jnp.float32),
            pltpu.SemaphoreType.DMA,
        ],
    )
    def sc_k(t1_hbm, t2_hbm, g_hbm, tt_hbm, g2_hbm, tp0_hbm, tp1_hbm, out_hbm,
             rt1_v, rt2_v, rg0_v, idx_vs, gat_vs, tp0_v, tp1_v,
             oute_v, outo_v, sem):
        wid = lax.axis_index("s") * nc + lax.axis_index("c")
        base = wid * bpw
        pltpu.sync_copy(t1_hbm.at[pl.ds(base, bpw)], rt1_v)
        pltpu.sync_copy(t2_hbm.at[pl.ds(base, bpw)], rt2_v)
        pltpu.sync_copy(g_hbm.at[pl.ds(base, bpw)], rg0_v)
        pltpu.sync_copy(tp0_hbm.at[pl.ds(base, bpw)], tp0_v)
        pltpu.sync_copy(tp1_hbm.at[pl.ds(base, bpw)], tp1_v)
        for k in range(bpw // 16):
            s = pl.ds(k * 16, 16)
            v1 = rt1_v[s] * 4
            v2 = rt2_v[s] * 4 + 2
            vg = rg0_v[s] * 2
            idx_vs[0][s] = v1
            idx_vs[1][s] = v1 + 1
            idx_vs[2][s] = v2
            idx_vs[3][s] = v2 + 1
            idx_vs[4][s] = vg
            idx_vs[5][s] = vg + 1
        copies = []
        for j in range(nchunks):
            c = pl.ds(j * 128, 128)
            for t in range(6):
                src = tt_hbm if t < 4 else g2_hbm
                copies.append(pltpu.async_copy(src.at[idx_vs[t].at[c]],
                                               gat_vs[t].at[c], sem))
        for c in copies:
            c.wait()
        for g in range(ngroups):
            s = pl.ds(g * 16, 16)
            oute_v[s] = gat_vs[0][s] + gat_vs[2][s] + gat_vs[4][s] + tp0_v[s]
            outo_v[s] = gat_vs[1][s] + gat_vs[3][s] + gat_vs[5][s] + tp1_v[s]
        pltpu.sync_copy(oute_v, out_hbm.at[pl.ds(base, bpw)])
        pltpu.sync_copy(outo_v, out_hbm.at[pl.ds(B + base, bpw)])

    return sc_k


def kernel(text1, text2, graph_data, scene_text, time_data, pretrained_emb,
           node_emb, time_w, time_b, W, b):
    tt, g2, tp0, tp1 = _fold(pretrained_emb, node_emb, time_data,
                             time_w.reshape(1, -1), time_b.reshape(1, -1),
                             W, b.reshape(1, -1))
    sc_k = _make_sc_gather()
    out_flat = sc_k(text1.astype(jnp.int32), text2.astype(jnp.int32),
                    graph_data.astype(jnp.int32),
                    tt.reshape(-1), g2.reshape(-1),
                    tp0.reshape(-1), tp1.reshape(-1))
    return out_flat.reshape(2, B).T


# speedup vs baseline: 2.1746x; 1.5239x over previous
import functools

import jax
import jax.numpy as jnp
from jax import lax
from jax.experimental import pallas as pl
from jax.experimental.pallas import tpu as pltpu
from jax.experimental.pallas import tpu_sc as plsc

B = 16384
VOCAB = 100000
NODE_NUM = 10000
TEXT_DIM = 64
TIME_DIM = 32

VPAD = 102400
NPAD = 10112
_FOLD_GRID = 10
_VBLK = VPAD // _FOLD_GRID

_INV_PI = 0.31830987334251404
_PI_HI = 3.1415927410125732
_PI_LO = -8.742277657347586e-08
_COS_COEFS = (1.9907545e-09, -2.7524447e-07, 2.4801026e-05, -0.0013888883,
              0.041666668, -0.5, 1.0)


def _fast_cos(x):
    k = jnp.floor(x * _INV_PI + 0.5)
    r = (x - k * _PI_HI) - k * _PI_LO
    r2 = r * r
    c = jnp.full_like(r2, _COS_COEFS[0])
    for cc in _COS_COEFS[1:]:
        c = c * r2 + cc
    odd = jnp.bitwise_and(k.astype(jnp.int32), 1)
    return jnp.where(odd == 0, c, -c)


def _fold_body(emb_ref, node_ref, time_ref, tw_ref, tb_ref, W_ref, b_ref,
               tt_ref, gg_ref, tp_ref):
    dn = (((1,), (1,)), ((), ()))
    z4 = jnp.zeros((4, TEXT_DIM), jnp.float32)
    wt8 = jnp.concatenate((W_ref[0:64, :].T, W_ref[64:128, :].T, z4), axis=0)
    tt_ref[...] = lax.dot_general(
        wt8, emb_ref[...], dn, preferred_element_type=jnp.float32)

    @pl.when(pl.program_id(0) == 0)
    def _():
        z6 = jnp.zeros((6, TEXT_DIM), jnp.float32)
        wg8 = jnp.concatenate((W_ref[128:192, :].T, z6), axis=0)
        gg_ref[:, 0:NODE_NUM] = lax.dot_general(
            wg8, node_ref[...], dn, preferred_element_type=jnp.float32)

        te = _fast_cos(time_ref[...] * tw_ref[...] + tb_ref[...])
        z6t = jnp.zeros((6, TIME_DIM), jnp.float32)
        w48 = jnp.concatenate((W_ref[192:224, :].T, z6t), axis=0)
        bcol = jnp.concatenate((b_ref[...].T, jnp.zeros((6, 1), jnp.float32)),
                               axis=0)
        tp_ref[...] = lax.dot_general(
            w48, te, dn, preferred_element_type=jnp.float32) + bcol


_fold = pl.pallas_call(
    _fold_body,
    grid=(_FOLD_GRID,),
    in_specs=[
        pl.BlockSpec((_VBLK, TEXT_DIM), lambda i: (i, 0)),
        pl.BlockSpec((NODE_NUM, TEXT_DIM), lambda i: (0, 0)),
        pl.BlockSpec((B, 1), lambda i: (0, 0)),
        pl.BlockSpec((1, TIME_DIM), lambda i: (0, 0)),
        pl.BlockSpec((1, TIME_DIM), lambda i: (0, 0)),
        pl.BlockSpec((2 * TEXT_DIM + TEXT_DIM + TIME_DIM, 2), lambda i: (0, 0)),
        pl.BlockSpec((1, 2), lambda i: (0, 0)),
    ],
    out_specs=[
        pl.BlockSpec((8, _VBLK), lambda i: (0, i)),
        pl.BlockSpec((8, NPAD), lambda i: (0, 0)),
        pl.BlockSpec((8, B), lambda i: (0, 0)),
    ],
    out_shape=[
        jax.ShapeDtypeStruct((8, VPAD), jnp.float32),
        jax.ShapeDtypeStruct((8, NPAD), jnp.float32),
        jax.ShapeDtypeStruct((8, B), jnp.float32),
    ],
)


@functools.lru_cache(maxsize=1)
def _make_sc_gather():
    info = plsc.get_sparse_core_info()
    nc, ns = info.num_cores, info.num_subcores
    nw = nc * ns
    bpw = B // nw
    nchunks = bpw // 128
    ngroups = bpw // 16
    mesh = plsc.VectorSubcoreMesh(core_axis_name="c", subcore_axis_name="s",
                                  num_cores=nc)

    @functools.partial(
        pl.kernel,
        mesh=mesh,
        out_type=jax.ShapeDtypeStruct((2 * B,), jnp.float32),
        scratch_types=[
            pltpu.VMEM((bpw,), jnp.int32),
            pltpu.VMEM((bpw,), jnp.int32),
            pltpu.VMEM((bpw,), jnp.int32),
            [pltpu.VMEM((bpw,), jnp.int32) for _ in range(4)],
            [pltpu.VMEM((bpw,), jnp.float32) for _ in range(6)],
            pltpu.VMEM((bpw,), jnp.float32),
            pltpu.VMEM((bpw,), jnp.float32),
            pltpu.VMEM((bpw,), jnp.float32),
            pltpu.VMEM((bpw,), jnp.float32),
            pltpu.SemaphoreType.DMA,
        ],
    )
    def sc_k(t1_hbm, t2_hbm, g_hbm, tt_hbm, gg_hbm, tp_hbm, out_hbm,
             rt1_v, rt2_v, rg0_v, idx_vs, gat_vs, tp0_v, tp1_v,
             oute_v, outo_v, sem):
        wid = lax.axis_index("s") * nc + lax.axis_index("c")
        base = wid * bpw
        pltpu.sync_copy(t1_hbm.at[pl.ds(base, bpw)], rt1_v)
        pltpu.sync_copy(t2_hbm.at[pl.ds(base, bpw)], rt2_v)
        pltpu.sync_copy(g_hbm.at[pl.ds(base, bpw)], rg0_v)
        pltpu.sync_copy(tp_hbm.at[pl.ds(base, bpw)], tp0_v)
        pltpu.sync_copy(tp_hbm.at[pl.ds(B + base, bpw)], tp1_v)
        for k in range(bpw // 16):
            s = pl.ds(k * 16, 16)
            idx_vs[0][s] = rt1_v[s] + VPAD
            idx_vs[1][s] = rt2_v[s] + 2 * VPAD
            idx_vs[2][s] = rt2_v[s] + 3 * VPAD
            idx_vs[3][s] = rg0_v[s] + NPAD
        srcs = (rt1_v, idx_vs[0], idx_vs[1], idx_vs[2], rg0_v, idx_vs[3])
        copies = []
        for j in range(nchunks):
            c = pl.ds(j * 128, 128)
            for t in range(6):
                tab = tt_hbm if t < 4 else gg_hbm
                copies.append(pltpu.async_copy(tab.at[srcs[t].at[c]],
                                               gat_vs[t].at[c], sem))
        for c in copies:
            c.wait()
        for g in range(ngroups):
            s = pl.ds(g * 16, 16)
            oute_v[s] = gat_vs[0][s] + gat_vs[2][s] + gat_vs[4][s] + tp0_v[s]
            outo_v[s] = gat_vs[1][s] + gat_vs[3][s] + gat_vs[5][s] + tp1_v[s]
        pltpu.sync_copy(oute_v, out_hbm.at[pl.ds(base, bpw)])
        pltpu.sync_copy(outo_v, out_hbm.at[pl.ds(B + base, bpw)])

    return sc_k


def kernel(text1, text2, graph_data, scene_text, time_data, pretrained_emb,
           node_emb, time_w, time_b, W, b):
    tt, gg, tp = _fold(pretrained_emb, node_emb, time_data,
                       time_w.reshape(1, -1), time_b.reshape(1, -1),
                       W, b.reshape(1, -1))
    sc_k = _make_sc_gather()
    out_flat = sc_k(text1.astype(jnp.int32), text2.astype(jnp.int32),
                    graph_data.astype(jnp.int32),
                    tt.reshape(-1), gg.reshape(-1), tp.reshape(-1))
    return out_flat.reshape(2, B).T


# speedup vs baseline: 4.8351x; 2.2235x over previous
import functools

import jax
import jax.numpy as jnp
from jax import lax
from jax.experimental import pallas as pl
from jax.experimental.pallas import tpu as pltpu
from jax.experimental.pallas import tpu_sc as plsc

B = 16384
VOCAB = 100000
NODE_NUM = 10000
TEXT_DIM = 64
TIME_DIM = 32

VPAD = 102400
NPAD = 10112
_FOLD_GRID = 10
_VBLK = VPAD // _FOLD_GRID

_INV_PI = 0.31830987334251404
_PI_HI = 3.1415927410125732
_PI_LO = -8.742277657347586e-08
_COS_COEFS = (1.9907545e-09, -2.7524447e-07, 2.4801026e-05, -0.0013888883,
              0.041666668, -0.5, 1.0)


def _fast_cos(x):
    k = jnp.floor(x * _INV_PI + 0.5)
    r = (x - k * _PI_HI) - k * _PI_LO
    r2 = r * r
    c = jnp.full_like(r2, _COS_COEFS[0])
    for cc in _COS_COEFS[1:]:
        c = c * r2 + cc
    odd = jnp.bitwise_and(k.astype(jnp.int32), 1)
    return jnp.where(odd == 0, c, -c)


def _fold_body(emb_ref, node_ref, time_ref, tw_ref, tb_ref, W_ref, b_ref,
               tt_ref, gg_ref, tp_ref):
    dn = (((1,), (0,)), ((), ()))
    z4 = jnp.zeros((4, TEXT_DIM), jnp.float32)
    wt8 = jnp.concatenate((W_ref[0:64, :].T, W_ref[64:128, :].T, z4), axis=0)
    tt_ref[...] = lax.dot_general(
        wt8, emb_ref[...], dn, preferred_element_type=jnp.float32)

    @pl.when(pl.program_id(0) == 0)
    def _():
        z6 = jnp.zeros((6, TEXT_DIM), jnp.float32)
        wg8 = jnp.concatenate((W_ref[128:192, :].T, z6), axis=0)
        gg_ref[:, 0:NODE_NUM] = lax.dot_general(
            wg8, node_ref[...], dn, preferred_element_type=jnp.float32)

        te = _fast_cos(tw_ref[...].T * time_ref[...] + tb_ref[...].T)
        z6t = jnp.zeros((6, TIME_DIM), jnp.float32)
        w48 = jnp.concatenate((W_ref[192:224, :].T, z6t), axis=0)
        bcol = jnp.concatenate((b_ref[...].T, jnp.zeros((6, 1), jnp.float32)),
                               axis=0)
        tp_ref[...] = lax.dot_general(
            w48, te, dn, preferred_element_type=jnp.float32) + bcol


_fold = pl.pallas_call(
    _fold_body,
    grid=(_FOLD_GRID,),
    in_specs=[
        pl.BlockSpec((TEXT_DIM, _VBLK), lambda i: (0, i)),
        pl.BlockSpec((TEXT_DIM, NODE_NUM), lambda i: (0, 0)),
        pl.BlockSpec((1, B), lambda i: (0, 0)),
        pl.BlockSpec((1, TIME_DIM), lambda i: (0, 0)),
        pl.BlockSpec((1, TIME_DIM), lambda i: (0, 0)),
        pl.BlockSpec((2 * TEXT_DIM + TEXT_DIM + TIME_DIM, 2), lambda i: (0, 0)),
        pl.BlockSpec((1, 2), lambda i: (0, 0)),
    ],
    out_specs=[
        pl.BlockSpec((8, _VBLK), lambda i: (0, i)),
        pl.BlockSpec((8, NPAD), lambda i: (0, 0)),
        pl.BlockSpec((8, B), lambda i: (0, 0)),
    ],
    out_shape=[
        jax.ShapeDtypeStruct((8, VPAD), jnp.float32),
        jax.ShapeDtypeStruct((8, NPAD), jnp.float32),
        jax.ShapeDtypeStruct((8, B), jnp.float32),
    ],
)


@functools.lru_cache(maxsize=1)
def _make_sc_gather():
    info = plsc.get_sparse_core_info()
    nc, ns = info.num_cores, info.num_subcores
    nw = nc * ns
    bpw = B // nw
    nchunks = bpw // 128
    ngroups = bpw // 16
    mesh = plsc.VectorSubcoreMesh(core_axis_name="c", subcore_axis_name="s",
                                  num_cores=nc)

    @functools.partial(
        pl.kernel,
        mesh=mesh,
        out_type=jax.ShapeDtypeStruct((2 * B,), jnp.float32),
        scratch_types=[
            pltpu.VMEM((bpw,), jnp.int32),
            pltpu.VMEM((bpw,), jnp.int32),
            pltpu.VMEM((bpw,), jnp.int32),
            [pltpu.VMEM((bpw,), jnp.int32) for _ in range(4)],
            [pltpu.VMEM((bpw,), jnp.float32) for _ in range(6)],
            pltpu.VMEM((bpw,), jnp.float32),
            pltpu.VMEM((bpw,), jnp.float32),
            pltpu.VMEM((bpw,), jnp.float32),
            pltpu.VMEM((bpw,), jnp.float32),
            pltpu.SemaphoreType.DMA,
        ],
    )
    def sc_k(t1_hbm, t2_hbm, g_hbm, tt_hbm, gg_hbm, tp_hbm, out_hbm,
             rt1_v, rt2_v, rg0_v, idx_vs, gat_vs, tp0_v, tp1_v,
             oute_v, outo_v, sem):
        wid = lax.axis_index("s") * nc + lax.axis_index("c")
        base = wid * bpw
        pltpu.sync_copy(t1_hbm.at[pl.ds(base, bpw)], rt1_v)
        pltpu.sync_copy(t2_hbm.at[pl.ds(base, bpw)], rt2_v)
        pltpu.sync_copy(g_hbm.at[pl.ds(base, bpw)], rg0_v)
        pltpu.sync_copy(tp_hbm.at[pl.ds(base, bpw)], tp0_v)
        pltpu.sync_copy(tp_hbm.at[pl.ds(B + base, bpw)], tp1_v)
        for k in range(bpw // 16):
            s = pl.ds(k * 16, 16)
            idx_vs[0][s] = rt1_v[s] + VPAD
            idx_vs[1][s] = rt2_v[s] + 2 * VPAD
            idx_vs[2][s] = rt2_v[s] + 3 * VPAD
            idx_vs[3][s] = rg0_v[s] + NPAD
        srcs = (rt1_v, idx_vs[0], idx_vs[1], idx_vs[2], rg0_v, idx_vs[3])
        copies = []
        for j in range(nchunks):
            c = pl.ds(j * 128, 128)
            for t in range(6):
                tab = tt_hbm if t < 4 else gg_hbm
                copies.append(pltpu.async_copy(tab.at[srcs[t].at[c]],
                                               gat_vs[t].at[c], sem))
        for c in copies:
            c.wait()
        for g in range(ngroups):
            s = pl.ds(g * 16, 16)
            oute_v[s] = gat_vs[0][s] + gat_vs[2][s] + gat_vs[4][s] + tp0_v[s]
            outo_v[s] = gat_vs[1][s] + gat_vs[3][s] + gat_vs[5][s] + tp1_v[s]
        pltpu.sync_copy(oute_v, out_hbm.at[pl.ds(base, bpw)])
        pltpu.sync_copy(outo_v, out_hbm.at[pl.ds(B + base, bpw)])

    return sc_k


def kernel(text1, text2, graph_data, scene_text, time_data, pretrained_emb,
           node_emb, time_w, time_b, W, b):
    tt, gg, tp = _fold(pretrained_emb.T, node_emb.T, time_data.T,
                       time_w.reshape(1, -1), time_b.reshape(1, -1),
                       W, b.reshape(1, -1))
    sc_k = _make_sc_gather()
    out_flat = sc_k(text1.astype(jnp.int32), text2.astype(jnp.int32),
                    graph_data.astype(jnp.int32),
                    tt.reshape(-1), gg.reshape(-1), tp.reshape(-1))
    return out_flat.reshape(2, B).T


# speedup vs baseline: 5.5550x; 1.1489x over previous
import functools

import jax
import jax.numpy as jnp
from jax import lax
from jax.experimental import pallas as pl
from jax.experimental.pallas import tpu as pltpu
from jax.experimental.pallas import tpu_sc as plsc

B = 16384
VOCAB = 100000
NODE_NUM = 10000
TEXT_DIM = 64
TIME_DIM = 32

VPAD = 102400
NPAD = 10112
_FOLD_GRID = 10
_VBLK = VPAD // _FOLD_GRID

_INV_PI = 0.31830987334251404
_PI_HI = 3.1415927410125732
_PI_LO = -8.742277657347586e-08
_COS_COEFS = (1.9907545e-09, -2.7524447e-07, 2.4801026e-05, -0.0013888883,
              0.041666668, -0.5, 1.0)


def _fast_cos(x):
    k = jnp.floor(x * _INV_PI + 0.5)
    r = (x - k * _PI_HI) - k * _PI_LO
    r2 = r * r
    c = jnp.full_like(r2, _COS_COEFS[0])
    for cc in _COS_COEFS[1:]:
        c = c * r2 + cc
    odd = jnp.bitwise_and(k.astype(jnp.int32), 1)
    return jnp.where(odd == 0, c, -c)


def _fold_body(emb_ref, node_ref, time_ref, tw_ref, tb_ref, WT_ref, b_ref,
               p0_ref, p1_ref, p2_ref, p3_ref, gg_ref, q0_ref, q1_ref):
    dn = (((1,), (0,)), ((), ()))
    z4 = jnp.zeros((4, TEXT_DIM), jnp.float32)
    wt8 = jnp.concatenate((WT_ref[:, 0:64], WT_ref[:, 64:128], z4), axis=0)
    res = lax.dot_general(
        wt8, emb_ref[...], dn, preferred_element_type=jnp.float32)
    p0_ref[...] = res[0].reshape(_VBLK // 128, 128)
    p1_ref[...] = res[1].reshape(_VBLK // 128, 128)
    p2_ref[...] = res[2].reshape(_VBLK // 128, 128)
    p3_ref[...] = res[3].reshape(_VBLK // 128, 128)

    @pl.when(pl.program_id(0) == 0)
    def _():
        z6 = jnp.zeros((6, TEXT_DIM), jnp.float32)
        wg8 = jnp.concatenate((WT_ref[:, 128:192], z6), axis=0)
        gg_ref[:, 0:NODE_NUM] = lax.dot_general(
            wg8, node_ref[...], dn, preferred_element_type=jnp.float32)

        te = _fast_cos(tw_ref[...].T * time_ref[...] + tb_ref[...].T)
        z6t = jnp.zeros((6, TIME_DIM), jnp.float32)
        w48 = jnp.concatenate((WT_ref[:, 192:224], z6t), axis=0)
        bcol = jnp.concatenate((b_ref[...].T, jnp.zeros((6, 1), jnp.float32)),
                               axis=0)
        tpres = lax.dot_general(
            w48, te, dn, preferred_element_type=jnp.float32) + bcol
        q0_ref[...] = tpres[0].reshape(B // 128, 128)
        q1_ref[...] = tpres[1].reshape(B // 128, 128)


_fold = pl.pallas_call(
    _fold_body,
    grid=(_FOLD_GRID,),
    in_specs=[
        pl.BlockSpec((TEXT_DIM, _VBLK), lambda i: (0, i)),
        pl.BlockSpec((TEXT_DIM, NODE_NUM), lambda i: (0, 0)),
        pl.BlockSpec((1, B), lambda i: (0, 0)),
        pl.BlockSpec((1, TIME_DIM), lambda i: (0, 0)),
        pl.BlockSpec((1, TIME_DIM), lambda i: (0, 0)),
        pl.BlockSpec((2, 2 * TEXT_DIM + TEXT_DIM + TIME_DIM), lambda i: (0, 0)),
        pl.BlockSpec((1, 2), lambda i: (0, 0)),
    ],
    out_specs=[
        pl.BlockSpec((_VBLK // 128, 128), lambda i: (i, 0)),
        pl.BlockSpec((_VBLK // 128, 128), lambda i: (i, 0)),
        pl.BlockSpec((_VBLK // 128, 128), lambda i: (i, 0)),
        pl.BlockSpec((_VBLK // 128, 128), lambda i: (i, 0)),
        pl.BlockSpec((8, NPAD), lambda i: (0, 0)),
        pl.BlockSpec((B // 128, 128), lambda i: (0, 0)),
        pl.BlockSpec((B // 128, 128), lambda i: (0, 0)),
    ],
    out_shape=[
        jax.ShapeDtypeStruct((VPAD // 128, 128), jnp.float32),
        jax.ShapeDtypeStruct((VPAD // 128, 128), jnp.float32),
        jax.ShapeDtypeStruct((VPAD // 128, 128), jnp.float32),
        jax.ShapeDtypeStruct((VPAD // 128, 128), jnp.float32),
        jax.ShapeDtypeStruct((8, NPAD), jnp.float32),
        jax.ShapeDtypeStruct((B // 128, 128), jnp.float32),
        jax.ShapeDtypeStruct((B // 128, 128), jnp.float32),
    ],
)


@functools.lru_cache(maxsize=1)
def _make_sc_gather():
    info = plsc.get_sparse_core_info()
    nc, ns = info.num_cores, info.num_subcores
    nw = nc * ns
    bpw = B // nw
    nchunks = bpw // 128
    ngroups = bpw // 16
    mesh = plsc.VectorSubcoreMesh(core_axis_name="c", subcore_axis_name="s",
                                  num_cores=nc)

    @functools.partial(
        pl.kernel,
        mesh=mesh,
        out_type=jax.ShapeDtypeStruct((2 * B,), jnp.float32),
        scratch_types=[
            pltpu.VMEM((bpw,), jnp.int32),
            pltpu.VMEM((bpw,), jnp.int32),
            pltpu.VMEM((bpw,), jnp.int32),
            pltpu.VMEM((bpw,), jnp.int32),
            [pltpu.VMEM((bpw,), jnp.float32) for _ in range(6)],
            pltpu.VMEM((bpw,), jnp.float32),
            pltpu.VMEM((bpw,), jnp.float32),
            pltpu.VMEM((bpw,), jnp.float32),
            pltpu.VMEM((bpw,), jnp.float32),
            pltpu.SemaphoreType.DMA,
        ],
    )
    def sc_k(t1_hbm, t2_hbm, g_hbm, p0_hbm, p1_hbm, p2_hbm, p3_hbm, gg_hbm,
             q0_hbm, q1_hbm, out_hbm,
             rt1_v, rt2_v, rg0_v, idxg_v, gat_vs, tp0_v, tp1_v,
             oute_v, outo_v, sem):
        wid = lax.axis_index("s") * nc + lax.axis_index("c")
        base = wid * bpw
        pltpu.sync_copy(t1_hbm.at[pl.ds(base, bpw)], rt1_v)
        pltpu.sync_copy(t2_hbm.at[pl.ds(base, bpw)], rt2_v)
        pltpu.sync_copy(g_hbm.at[pl.ds(base, bpw)], rg0_v)
        pltpu.sync_copy(q0_hbm.at[pl.ds(base, bpw)], tp0_v)
        pltpu.sync_copy(q1_hbm.at[pl.ds(base, bpw)], tp1_v)
        for k in range(bpw // 16):
            s = pl.ds(k * 16, 16)
            idxg_v[s] = rg0_v[s] + NPAD
        srcs = ((p0_hbm, rt1_v), (p1_hbm, rt1_v), (p2_hbm, rt2_v),
                (p3_hbm, rt2_v), (gg_hbm, rg0_v), (gg_hbm, idxg_v))
        copies = []
        for j in range(nchunks):
            c = pl.ds(j * 128, 128)
            for t, (tab, idx) in enumerate(srcs):
                copies.append(pltpu.async_copy(tab.at[idx.at[c]],
                                               gat_vs[t].at[c], sem))
        for c in copies:
            c.wait()
        for g in range(ngroups):
            s = pl.ds(g * 16, 16)
            oute_v[s] = gat_vs[0][s] + gat_vs[2][s] + gat_vs[4][s] + tp0_v[s]
            outo_v[s] = gat_vs[1][s] + gat_vs[3][s] + gat_vs[5][s] + tp1_v[s]
        pltpu.sync_copy(oute_v, out_hbm.at[pl.ds(base, bpw)])
        pltpu.sync_copy(outo_v, out_hbm.at[pl.ds(B + base, bpw)])

    return sc_k


def kernel(text1, text2, graph_data, scene_text, time_data, pretrained_emb,
           node_emb, time_w, time_b, W, b):
    p0, p1, p2, p3, gg, q0, q1 = _fold(
        pretrained_emb.T, node_emb.T, time_data.T,
        time_w.reshape(1, -1), time_b.reshape(1, -1),
        W.T, b.reshape(1, -1))
    sc_k = _make_sc_gather()
    out_flat = sc_k(text1.astype(jnp.int32), text2.astype(jnp.int32),
                    graph_data.astype(jnp.int32),
                    p0.reshape(-1), p1.reshape(-1), p2.reshape(-1),
                    p3.reshape(-1), gg.reshape(-1),
                    q0.reshape(-1), q1.reshape(-1))
    return out_flat.reshape(2, B).T


# speedup vs baseline: 5.7689x; 1.0385x over previous
import functools

import jax
import jax.numpy as jnp
from jax import lax
from jax.experimental import pallas as pl
from jax.experimental.pallas import tpu as pltpu
from jax.experimental.pallas import tpu_sc as plsc

B = 16384
VOCAB = 100000
NODE_NUM = 10000
TEXT_DIM = 64
TIME_DIM = 32

VPAD = 102400
NPAD = 10112
_FOLD_GRID = 10
_VBLK = VPAD // _FOLD_GRID

_INV_PI = 0.31830987334251404
_PI_HI = 3.1415927410125732
_PI_LO = -8.742277657347586e-08
_COS_COEFS = (1.9907545e-09, -2.7524447e-07, 2.4801026e-05, -0.0013888883,
              0.041666668, -0.5, 1.0)


def _fast_cos(x):
    k = jnp.floor(x * _INV_PI + 0.5)
    r = (x - k * _PI_HI) - k * _PI_LO
    r2 = r * r
    c = jnp.full_like(r2, _COS_COEFS[0])
    for cc in _COS_COEFS[1:]:
        c = c * r2 + cc
    odd = jnp.bitwise_and(k.astype(jnp.int32), 1)
    return jnp.where(odd == 0, c, -c)


def _fold_body(emb_ref, node_ref, time_ref, tw_ref, tb_ref, WT_ref, b_ref,
               p0_ref, p1_ref, p2_ref, p3_ref, gg_ref, q0_ref, q1_ref):
    dn = (((1,), (0,)), ((), ()))
    z4 = jnp.zeros((4, TEXT_DIM), jnp.float32)
    wt8 = jnp.concatenate((WT_ref[:, 0:64], WT_ref[:, 64:128], z4), axis=0)
    res = lax.dot_general(
        wt8, emb_ref[...], dn, preferred_element_type=jnp.float32)
    p0_ref[...] = res[0].reshape(_VBLK // 128, 128)
    p1_ref[...] = res[1].reshape(_VBLK // 128, 128)
    p2_ref[...] = res[2].reshape(_VBLK // 128, 128)
    p3_ref[...] = res[3].reshape(_VBLK // 128, 128)

    @pl.when(pl.program_id(0) == 0)
    def _():
        z6 = jnp.zeros((6, TEXT_DIM), jnp.float32)
        wg8 = jnp.concatenate((WT_ref[:, 128:192], z6), axis=0)
        gg_ref[:, 0:NODE_NUM] = lax.dot_general(
            wg8, node_ref[...], dn, preferred_element_type=jnp.float32)

        te = _fast_cos(tw_ref[...].T * time_ref[...] + tb_ref[...].T)
        z6t = jnp.zeros((6, TIME_DIM), jnp.float32)
        w48 = jnp.concatenate((WT_ref[:, 192:224], z6t), axis=0)
        bcol = jnp.concatenate((b_ref[...].T, jnp.zeros((6, 1), jnp.float32)),
                               axis=0)
        tpres = lax.dot_general(
            w48, te, dn, preferred_element_type=jnp.float32) + bcol
        q0_ref[...] = tpres[0].reshape(B // 128, 128)
        q1_ref[...] = tpres[1].reshape(B // 128, 128)


_fold = pl.pallas_call(
    _fold_body,
    grid=(_FOLD_GRID,),
    in_specs=[
        pl.BlockSpec((TEXT_DIM, _VBLK), lambda i: (0, i)),
        pl.BlockSpec((TEXT_DIM, NODE_NUM), lambda i: (0, 0)),
        pl.BlockSpec((1, B), lambda i: (0, 0)),
        pl.BlockSpec((1, TIME_DIM), lambda i: (0, 0)),
        pl.BlockSpec((1, TIME_DIM), lambda i: (0, 0)),
        pl.BlockSpec((2, 2 * TEXT_DIM + TEXT_DIM + TIME_DIM), lambda i: (0, 0)),
        pl.BlockSpec((1, 2), lambda i: (0, 0)),
    ],
    out_specs=[
        pl.BlockSpec((_VBLK // 128, 128), lambda i: (i, 0)),
        pl.BlockSpec((_VBLK // 128, 128), lambda i: (i, 0)),
        pl.BlockSpec((_VBLK // 128, 128), lambda i: (i, 0)),
        pl.BlockSpec((_VBLK // 128, 128), lambda i: (i, 0)),
        pl.BlockSpec((8, NPAD), lambda i: (0, 0)),
        pl.BlockSpec((B // 128, 128), lambda i: (0, 0)),
        pl.BlockSpec((B // 128, 128), lambda i: (0, 0)),
    ],
    out_shape=[
        jax.ShapeDtypeStruct((VPAD // 128, 128), jnp.float32),
        jax.ShapeDtypeStruct((VPAD // 128, 128), jnp.float32),
        jax.ShapeDtypeStruct((VPAD // 128, 128), jnp.float32),
        jax.ShapeDtypeStruct((VPAD // 128, 128), jnp.float32),
        jax.ShapeDtypeStruct((8, NPAD), jnp.float32),
        jax.ShapeDtypeStruct((B // 128, 128), jnp.float32),
        jax.ShapeDtypeStruct((B // 128, 128), jnp.float32),
    ],
)


@functools.lru_cache(maxsize=1)
def _make_sc_gather():
    info = plsc.get_sparse_core_info()
    nc, ns = info.num_cores, info.num_subcores
    nw = nc * ns
    bpw = B // nw
    nchunks = bpw // 128
    ngroups = bpw // 16
    mesh = plsc.VectorSubcoreMesh(core_axis_name="c", subcore_axis_name="s",
                                  num_cores=nc)

    @functools.partial(
        pl.kernel,
        mesh=mesh,
        out_type=jax.ShapeDtypeStruct((2 * B,), jnp.float32),
        scratch_types=[
            pltpu.VMEM((bpw,), jnp.int32),
            pltpu.VMEM((bpw,), jnp.int32),
            pltpu.VMEM((bpw,), jnp.int32),
            pltpu.VMEM((bpw,), jnp.int32),
            [pltpu.VMEM((bpw,), jnp.float32) for _ in range(6)],
            pltpu.VMEM((bpw,), jnp.float32),
            pltpu.VMEM((bpw,), jnp.float32),
            pltpu.VMEM((bpw,), jnp.float32),
            pltpu.VMEM((bpw,), jnp.float32),
            pltpu.SemaphoreType.DMA,
        ],
    )
    def sc_k(t1_hbm, t2_hbm, g_hbm, p0_hbm, p1_hbm, p2_hbm, p3_hbm, gg_hbm,
             q0_hbm, q1_hbm, out_hbm,
             rt1_v, rt2_v, rg0_v, idxg_v, gat_vs, tp0_v, tp1_v,
             oute_v, outo_v, sem):
        wid = lax.axis_index("s") * nc + lax.axis_index("c")
        base = wid * bpw
        s_in = pl.ds(base, bpw)
        loads = [pltpu.async_copy(t1_hbm.at[s_in], rt1_v, sem),
                 pltpu.async_copy(t2_hbm.at[s_in], rt2_v, sem),
                 pltpu.async_copy(g_hbm.at[s_in], rg0_v, sem),
                 pltpu.async_copy(q0_hbm.at[s_in], tp0_v, sem),
                 pltpu.async_copy(q1_hbm.at[s_in], tp1_v, sem)]
        for c in loads:
            c.wait()
        for k in range(bpw // 16):
            s = pl.ds(k * 16, 16)
            idxg_v[s] = rg0_v[s] + NPAD
        srcs = ((p0_hbm, rt1_v), (p1_hbm, rt1_v), (p2_hbm, rt2_v),
                (p3_hbm, rt2_v), (gg_hbm, rg0_v), (gg_hbm, idxg_v))
        copies = []
        for j in range(nchunks):
            c = pl.ds(j * 128, 128)
            for t, (tab, idx) in enumerate(srcs):
                copies.append(pltpu.async_copy(tab.at[idx.at[c]],
                                               gat_vs[t].at[c], sem))
        for c in copies:
            c.wait()
        for g in range(ngroups):
            s = pl.ds(g * 16, 16)
            oute_v[s] = gat_vs[0][s] + gat_vs[2][s] + gat_vs[4][s] + tp0_v[s]
            outo_v[s] = gat_vs[1][s] + gat_vs[3][s] + gat_vs[5][s] + tp1_v[s]
        st0 = pltpu.async_copy(oute_v, out_hbm.at[pl.ds(base, bpw)], sem)
        st1 = pltpu.async_copy(outo_v, out_hbm.at[pl.ds(B + base, bpw)], sem)
        st0.wait()
        st1.wait()

    return sc_k


def kernel(text1, text2, graph_data, scene_text, time_data, pretrained_emb,
           node_emb, time_w, time_b, W, b):
    p0, p1, p2, p3, gg, q0, q1 = _fold(
        pretrained_emb.T, node_emb.T, time_data.T,
        time_w.reshape(1, -1), time_b.reshape(1, -1),
        W.T, b.reshape(1, -1))
    sc_k = _make_sc_gather()
    out_flat = sc_k(text1.astype(jnp.int32), text2.astype(jnp.int32),
                    graph_data.astype(jnp.int32),
                    p0.reshape(-1), p1.reshape(-1), p2.reshape(-1),
                    p3.reshape(-1), gg.reshape(-1),
                    q0.reshape(-1), q1.reshape(-1))
    return out_flat.reshape(2, B).T


# speedup vs baseline: 6.0082x; 1.0415x over previous
import functools

import jax
import jax.numpy as jnp
from jax import lax
from jax.experimental import pallas as pl
from jax.experimental.pallas import tpu as pltpu
from jax.experimental.pallas import tpu_sc as plsc

B = 16384
VOCAB = 100000
NODE_NUM = 10000
TEXT_DIM = 64
TIME_DIM = 32

VPAD = 102400
NPAD = 10240
_FOLD_GRID = 10
_VBLK = VPAD // _FOLD_GRID

_INV_PI = 0.31830987334251404
_PI_HI = 3.1415927410125732
_PI_LO = -8.742277657347586e-08
_COS_COEFS = (1.9907545e-09, -2.7524447e-07, 2.4801026e-05, -0.0013888883,
              0.041666668, -0.5, 1.0)


def _fast_cos(x):
    k = jnp.floor(x * _INV_PI + 0.5)
    r = (x - k * _PI_HI) - k * _PI_LO
    r2 = r * r
    c = jnp.full_like(r2, _COS_COEFS[0])
    for cc in _COS_COEFS[1:]:
        c = c * r2 + cc
    odd = jnp.bitwise_and(k.astype(jnp.int32), 1)
    return jnp.where(odd == 0, c, -c)


def _fold_body(emb_ref, node_ref, time_ref, tw_ref, tb_ref, WT_ref, b_ref,
               p0_ref, p1_ref, p2_ref, p3_ref, g0_ref, g1_ref, q0_ref, q1_ref):
    dn = (((1,), (0,)), ((), ()))
    z4 = jnp.zeros((4, TEXT_DIM), jnp.float32)
    wt8 = jnp.concatenate((WT_ref[:, 0:64], WT_ref[:, 64:128], z4), axis=0)
    res = lax.dot_general(
        wt8, emb_ref[...], dn, preferred_element_type=jnp.float32)
    p0_ref[...] = res[0].reshape(_VBLK // 128, 128)
    p1_ref[...] = res[1].reshape(_VBLK // 128, 128)
    p2_ref[...] = res[2].reshape(_VBLK // 128, 128)
    p3_ref[...] = res[3].reshape(_VBLK // 128, 128)

    @pl.when(pl.program_id(0) == 0)
    def _():
        z6 = jnp.zeros((6, TEXT_DIM), jnp.float32)
        wg8 = jnp.concatenate((WT_ref[:, 128:192], z6), axis=0)
        gres = lax.dot_general(
            wg8, node_ref[...], dn, preferred_element_type=jnp.float32)
        g0_ref[...] = gres[0].reshape(NPAD // 128, 128)
        g1_ref[...] = gres[1].reshape(NPAD // 128, 128)

        te = _fast_cos(tw_ref[...].T * time_ref[...] + tb_ref[...].T)
        z6t = jnp.zeros((6, TIME_DIM), jnp.float32)
        w48 = jnp.concatenate((WT_ref[:, 192:224], z6t), axis=0)
        bcol = jnp.concatenate((b_ref[...].T, jnp.zeros((6, 1), jnp.float32)),
                               axis=0)
        tpres = lax.dot_general(
            w48, te, dn, preferred_element_type=jnp.float32) + bcol
        q0_ref[...] = tpres[0].reshape(B // 128, 128)
        q1_ref[...] = tpres[1].reshape(B // 128, 128)


_fold = pl.pallas_call(
    _fold_body,
    grid=(_FOLD_GRID,),
    in_specs=[
        pl.BlockSpec((TEXT_DIM, _VBLK), lambda i: (0, i)),
        pl.BlockSpec((TEXT_DIM, NPAD), lambda i: (0, 0)),
        pl.BlockSpec((1, B), lambda i: (0, 0)),
        pl.BlockSpec((1, TIME_DIM), lambda i: (0, 0)),
        pl.BlockSpec((1, TIME_DIM), lambda i: (0, 0)),
        pl.BlockSpec((2, 2 * TEXT_DIM + TEXT_DIM + TIME_DIM), lambda i: (0, 0)),
        pl.BlockSpec((1, 2), lambda i: (0, 0)),
    ],
    out_specs=[
        pl.BlockSpec((_VBLK // 128, 128), lambda i: (i, 0)),
        pl.BlockSpec((_VBLK // 128, 128), lambda i: (i, 0)),
        pl.BlockSpec((_VBLK // 128, 128), lambda i: (i, 0)),
        pl.BlockSpec((_VBLK // 128, 128), lambda i: (i, 0)),
        pl.BlockSpec((NPAD // 128, 128), lambda i: (0, 0)),
        pl.BlockSpec((NPAD // 128, 128), lambda i: (0, 0)),
        pl.BlockSpec((B // 128, 128), lambda i: (0, 0)),
        pl.BlockSpec((B // 128, 128), lambda i: (0, 0)),
    ],
    out_shape=[
        jax.ShapeDtypeStruct((VPAD // 128, 128), jnp.float32),
        jax.ShapeDtypeStruct((VPAD // 128, 128), jnp.float32),
        jax.ShapeDtypeStruct((VPAD // 128, 128), jnp.float32),
        jax.ShapeDtypeStruct((VPAD // 128, 128), jnp.float32),
        jax.ShapeDtypeStruct((NPAD // 128, 128), jnp.float32),
        jax.ShapeDtypeStruct((NPAD // 128, 128), jnp.float32),
        jax.ShapeDtypeStruct((B // 128, 128), jnp.float32),
        jax.ShapeDtypeStruct((B // 128, 128), jnp.float32),
    ],
)


@functools.lru_cache(maxsize=1)
def _make_sc_gather():
    info = plsc.get_sparse_core_info()
    nc, ns = info.num_cores, info.num_subcores
    nw = nc * ns
    bpw = B // nw
    nchunks = bpw // 128
    ngroups = bpw // 16
    mesh = plsc.VectorSubcoreMesh(core_axis_name="c", subcore_axis_name="s",
                                  num_cores=nc)

    @functools.partial(
        pl.kernel,
        mesh=mesh,
        out_type=jax.ShapeDtypeStruct((2 * B,), jnp.float32),
        scratch_types=[
            pltpu.VMEM((bpw,), jnp.int32),
            pltpu.VMEM((bpw,), jnp.int32),
            pltpu.VMEM((bpw,), jnp.int32),
            [pltpu.VMEM((bpw,), jnp.float32) for _ in range(6)],
            pltpu.VMEM((bpw,), jnp.float32),
            pltpu.VMEM((bpw,), jnp.float32),
            pltpu.VMEM((bpw,), jnp.float32),
            pltpu.VMEM((bpw,), jnp.float32),
            pltpu.SemaphoreType.DMA,
        ],
    )
    def sc_k(t1_hbm, t2_hbm, g_hbm, p0_hbm, p1_hbm, p2_hbm, p3_hbm,
             g0_hbm, g1_hbm, q0_hbm, q1_hbm, out_hbm,
             rt1_v, rt2_v, rg0_v, gat_vs, tp0_v, tp1_v,
             oute_v, outo_v, sem):
        wid = lax.axis_index("s") * nc + lax.axis_index("c")
        base = wid * bpw
        s_in = pl.ds(base, bpw)
        loads = [pltpu.async_copy(t1_hbm.at[s_in], rt1_v, sem),
                 pltpu.async_copy(t2_hbm.at[s_in], rt2_v, sem),
                 pltpu.async_copy(g_hbm.at[s_in], rg0_v, sem),
                 pltpu.async_copy(q0_hbm.at[s_in], tp0_v, sem),
                 pltpu.async_copy(q1_hbm.at[s_in], tp1_v, sem)]
        for c in loads:
            c.wait()
        srcs = ((p0_hbm, rt1_v), (p1_hbm, rt1_v), (p2_hbm, rt2_v),
                (p3_hbm, rt2_v), (g0_hbm, rg0_v), (g1_hbm, rg0_v))
        copies = []
        for j in range(nchunks):
            c = pl.ds(j * 128, 128)
            for t, (tab, idx) in enumerate(srcs):
                copies.append(pltpu.async_copy(tab.at[idx.at[c]],
                                               gat_vs[t].at[c], sem))
        for c in copies:
            c.wait()
        for g in range(ngroups):
            s = pl.ds(g * 16, 16)
            oute_v[s] = gat_vs[0][s] + gat_vs[2][s] + gat_vs[4][s] + tp0_v[s]
            outo_v[s] = gat_vs[1][s] + gat_vs[3][s] + gat_vs[5][s] + tp1_v[s]
        st0 = pltpu.async_copy(oute_v, out_hbm.at[pl.ds(base, bpw)], sem)
        st1 = pltpu.async_copy(outo_v, out_hbm.at[pl.ds(B + base, bpw)], sem)
        st0.wait()
        st1.wait()

    return sc_k


def kernel(text1, text2, graph_data, scene_text, time_data, pretrained_emb,
           node_emb, time_w, time_b, W, b):
    p0, p1, p2, p3, g0, g1, q0, q1 = _fold(
        pretrained_emb.T, node_emb.T, time_data.T,
        time_w.reshape(1, -1), time_b.reshape(1, -1),
        W.T, b.reshape(1, -1))
    sc_k = _make_sc_gather()
    out_flat = sc_k(text1.astype(jnp.int32), text2.astype(jnp.int32),
                    graph_data.astype(jnp.int32),
                    p0.reshape(-1), p1.reshape(-1), p2.reshape(-1),
                    p3.reshape(-1), g0.reshape(-1), g1.reshape(-1),
                    q0.reshape(-1), q1.reshape(-1))
    return out_flat.reshape(2, B).T
